# baseline (speedup 1.0000x reference)
import functools

import jax
import jax.numpy as jnp
from jax import lax
from jax.experimental import pallas as pl
from jax.experimental.pallas import tpu as pltpu
from jax.experimental.pallas import tpu_sc as plsc

H = 128
CE = 64
NP = 10240
EP = 327680
NSPLIT = 4
EH = EP // NSPLIT
TE = 2048
TN = 1024

NW = 32
GC = 64
GR = 4
SC = 128

_INTERPRET = False


def _sc_mesh():
    return plsc.VectorSubcoreMesh(core_axis_name="c", subcore_axis_name="s")


def _sc_gather_add(a, b, src, dst):
    eph = src.shape[0]
    epw = eph // NW
    gm = epw // GC

    @functools.partial(
        pl.kernel, mesh=_sc_mesh(),
        out_type=jax.ShapeDtypeStruct((eph, H), jnp.float32),
        scratch_types=[
            pltpu.VMEM((epw,), jnp.int32),
            pltpu.VMEM((epw,), jnp.int32),
        ] + [pltpu.VMEM((GC, H), jnp.float32)] * (2 * GR)
          + [pltpu.SemaphoreType.DMA] * (2 * GR),
    )
    def k(a_hbm, b_hbm, src_hbm, dst_hbm, g_hbm, si, di, *bufs):
        bas = bufs[0:GR]
        bbs = bufs[GR:2 * GR]
        gsem = bufs[2 * GR:3 * GR]
        ssem = bufs[3 * GR:4 * GR]
        wid = lax.axis_index("s") * 2 + lax.axis_index("c")
        base = wid * epw
        pltpu.sync_copy(src_hbm.at[pl.ds(base, epw)], si)
        pltpu.sync_copy(dst_hbm.at[pl.ds(base, epw)], di)

        def fire(m, q):
            pltpu.make_async_copy(
                a_hbm.at[si.at[pl.ds(m * GC, GC)]], bas[q], gsem[q]).start()
            pltpu.make_async_copy(
                b_hbm.at[di.at[pl.ds(m * GC, GC)]], bbs[q], gsem[q]).start()

        def gwait(m, q):
            pltpu.make_async_copy(
                a_hbm.at[si.at[pl.ds(m * GC, GC)]], bas[q], gsem[q]).wait()
            pltpu.make_async_copy(
                b_hbm.at[di.at[pl.ds(m * GC, GC)]], bbs[q], gsem[q]).wait()

        def store_desc(m, q):
            return pltpu.make_async_copy(
                bas[q], g_hbm.at[pl.ds(base + m * GC, GC), :], ssem[q])

        for mm in range(2):
            fire(mm, mm)

        def outer(o, carry):
            for p in range(GR):
                m = o * GR + p
                q = (p + 2) % GR

                @pl.when(m >= 2)
                def _():
                    store_desc(m - 2, q).wait()

                @pl.when(m + 2 < gm)
                def _():
                    fire(m + 2, q)

                gwait(m, p)

                @plsc.parallel_loop(0, GC, 1, unroll=4)
                def radd(r, _p=p):
                    for j in range(H // 16):
                        s = pl.ds(j * 16, 16)
                        bas[_p][r, s] = bas[_p][r, s] + bbs[_p][r, s]

                store_desc(m, p).start()
            return carry

        lax.fori_loop(0, gm // GR, outer, 0)
        for mm in (gm - 2, gm - 1):
            store_desc(mm, mm % GR).wait()

    return k(a, b, src, dst)


def _sc_scatter_add(h, dst3, zeros_hbm):
    eph = h.shape[0]
    epw = eph // NW
    sm = epw // SC

    @functools.partial(
        pl.kernel, mesh=_sc_mesh(),
        out_type=jax.ShapeDtypeStruct((2, NP, H), jnp.float32),
        scratch_types=[
            pltpu.VMEM((sm, SC), jnp.int32),
            pltpu.VMEM((SC, H), jnp.float32),
            pltpu.VMEM((SC, H), jnp.float32),
            pltpu.VMEM_SHARED((NP, H), jnp.float32),
            pltpu.SemaphoreType.DMA,
            pltpu.SemaphoreType.DMA,
            pltpu.SemaphoreType.DMA,
            pltpu.SemaphoreType.DMA,
        ],
    )
    def k(h_hbm, dst_hbm, z_hbm, out_hbm, di2, bh0, bh1, acc, hs0, hs1, ss0, ss1):
        bhs = (bh0, bh1)
        hsem = (hs0, hs1)
        ssem = (ss0, ss1)
        ci = lax.axis_index("c")
        sj = lax.axis_index("s")
        wid = sj * 2 + ci
        rows = NP // 16
        r0 = sj * rows
        pltpu.sync_copy(z_hbm.at[pl.ds(r0, rows), :], acc.at[pl.ds(r0, rows), :])
        pltpu.sync_copy(dst_hbm.at[wid], di2)
        base = wid * epw

        def hload_desc(m, q):
            return pltpu.make_async_copy(
                h_hbm.at[pl.ds(base + m * SC, SC), :], bhs[q], hsem[q])

        def sadd_desc(m, q):
            return pltpu.make_async_copy(bhs[q], acc.at[di2.at[m]], ssem[q])

        plsc.subcore_barrier()
        hload_desc(0, 0).start()

        def outer(o, carry):
            for p in range(2):
                m = o * 2 + p

                @pl.when(m >= 1)
                def _():
                    sadd_desc(m - 1, 1 - p).wait()

                @pl.when(m + 1 < sm)
                def _():
                    hload_desc(m + 1, 1 - p).start()

                hload_desc(m, p).wait()
                sadd_desc(m, p).start(add=True)
            return carry

        lax.fori_loop(0, sm // 2, outer, 0)
        sadd_desc(sm - 1, 1).wait()
        plsc.subcore_barrier()
        pltpu.sync_copy(acc.at[pl.ds(r0, rows), :],
                        out_hbm.at[ci, pl.ds(r0, rows), :])

    return k(h, dst3, zeros_hbm)


def _dot(a, b):
    return jnp.dot(a, b, preferred_element_type=jnp.float32)


def _encode_body(lg_ref, gw1_ref, gb1_ref, nf_ref):
    nfh = jnp.maximum(_dot(lg_ref[...], gw1_ref[...]) + gb1_ref[...], 0.0)
    nf_ref[...] = jnp.concatenate([nfh, jnp.zeros_like(nfh)], axis=-1)


def _ab_body(nf_ref, ws_ref, wd_ref, a_ref, b_ref):
    nf = nf_ref[...]
    a_ref[...] = _dot(nf, ws_ref[...])
    b_ref[...] = _dot(nf, wd_ref[...])


def _edge_body(g_ref, ea_ref, ew1_ref, eb1_ref, ew2_ref, eb2_ref,
               me_ref, mb1_ref, mw2_ref, mb2_ref, h_ref):
    ef = jnp.maximum(_dot(ea_ref[...], ew1_ref[...]) + eb1_ref[...], 0.0)
    ef = _dot(ef, ew2_ref[...]) + eb2_ref[...]
    pre = g_ref[...] + _dot(ef, me_ref[...]) + mb1_ref[...]
    h_ref[...] = _dot(jnp.maximum(pre, 0.0), mw2_ref[...]) + mb2_ref[...]


def _node_body(nf_ref, g0_ref, g1_ref, g2_ref, g3_ref, g4_ref, g5_ref,
               g6_ref, g7_ref, uwn_ref, uwa_ref,
               ub1_ref, uw2_ref, ub2_ref, lng_ref, lnb_ref, out_ref):
    nf = nf_ref[...]
    agg = ((g0_ref[...] + g1_ref[...]) + (g2_ref[...] + g3_ref[...]) +
           (g4_ref[...] + g5_ref[...]) + (g6_ref[...] + g7_ref[...]))
    pre = _dot(nf, uwn_ref[...]) + _dot(agg, uwa_ref[...]) + ub1_ref[...]
    upd = _dot(jnp.maximum(pre, 0.0), uw2_ref[...]) + ub2_ref[...]
    x = nf + upd
    mean = jnp.mean(x, axis=-1, keepdims=True)
    var = jnp.mean((x - mean) ** 2, axis=-1, keepdims=True)
    out_ref[...] = (x - mean) * jax.lax.rsqrt(var + 1e-5) * lng_ref[...] + lnb_ref[...]


def _out_body(nf_ref, ow1_ref, ob1_ref, ow2_ref, ob2_ref, out_ref):
    h = jnp.maximum(_dot(nf_ref[...], ow1_ref[...]) + ob1_ref[...], 0.0)
    out_ref[...] = _dot(h, ow2_ref[...]) + ob2_ref[...]


def _row_spec(t, k):
    return pl.BlockSpec((t, k), lambda i: (i, 0))


def _full_spec(shape):
    return pl.BlockSpec(shape, lambda i: (0,) * len(shape))


def _tc_call(body, nrows, trows, ins, out_shapes, out_dtype=jnp.float32):
    in_specs = []
    for x in ins:
        if x.ndim == 2 and x.shape[0] == nrows:
            in_specs.append(_row_spec(trows, x.shape[1]))
        else:
            in_specs.append(_full_spec(x.shape))
    if isinstance(out_shapes, tuple):
        out_shape = tuple(jax.ShapeDtypeStruct((nrows, k), out_dtype)
                          for k in out_shapes)
        out_specs = tuple(_row_spec(trows, k) for k in out_shapes)
    else:
        out_shape = jax.ShapeDtypeStruct((nrows, out_shapes), out_dtype)
        out_specs = _row_spec(trows, out_shapes)
    return pl.pallas_call(
        body,
        grid=(nrows // trows,),
        in_specs=in_specs,
        out_specs=out_specs,
        out_shape=out_shape,
        interpret=_INTERPRET,
    )(*ins)


def _row(v):
    return v.reshape(1, -1)


def kernel(edge_index, edge_attr, local_geometry, num_nodes, params):
    p = params
    N = local_geometry.shape[0]
    E = edge_attr.shape[0]

    src = edge_index[0].astype(jnp.int32)
    dst = edge_index[1].astype(jnp.int32)
    src_p = jnp.concatenate([src, jnp.zeros((EP - E,), jnp.int32)])
    dst_p = jnp.concatenate([dst, jnp.full((EP - E,), N, jnp.int32)])
    ea_p = jnp.zeros((EP, 8), jnp.float32).at[:E, :3].set(edge_attr)
    lg_p = jnp.zeros((NP, 8), jnp.float32).at[:N, :5].set(local_geometry)

    gw1 = jnp.zeros((8, H // 2), jnp.float32).at[:5].set(p['gw1'])
    ew1 = jnp.zeros((8, CE), jnp.float32).at[:3].set(p['ew1'])

    nf = _tc_call(_encode_body, NP, TN, (lg_p, gw1, _row(p['gb1'])), H)
    zeros_np = jnp.zeros((NP, H), jnp.float32)

    halves = []
    for hi in range(NSPLIT):
        sl = slice(hi * EH, (hi + 1) * EH)
        halves.append((src_p[sl], dst_p[sl],
                       dst_p[sl].reshape(NW, EH // NW // SC, SC), ea_p[sl]))

    for li, L in enumerate(p['layers']):
        ws, wd, me = L['mw1'][:H], L['mw1'][H:2 * H], L['mw1'][2 * H:]
        a, b = _tc_call(_ab_body, NP, TN, (nf, ws, wd), (H, H))

        gs = [_sc_gather_add(a, b, sh, dh) for sh, dh, _, _ in halves]
        hs = [_tc_call(_edge_body, EH, TE,
                       (g, eah, ew1, _row(p['eb1']), p['ew2'], _row(p['eb2']),
                        me, _row(L['mb1']), L['mw2'], _row(L['mb2'])), H)
              for g, (_, _, _, eah) in zip(gs, halves)]
        ps = [_sc_scatter_add(h, d3, zeros_np)
              for h, (_, _, d3, _) in zip(hs, halves)]

        aggs = tuple(pp[i] for pp in ps for i in range(2))
        nf = _tc_call(_node_body, NP, TN,
                      (nf,) + aggs +
                      (L['uw1'][:H], L['uw1'][H:], _row(L['ub1']),
                       L['uw2'], _row(L['ub2']),
                       _row(L['ln_g']), _row(L['ln_b'])), H)

    out = _tc_call(_out_body, NP, TN,
                   (nf, p['ow1'], _row(p['ob1']), p['ow2'], _row(p['ob2'])), H)
    return out[:N]


# speedup vs baseline: 1.0552x; 1.0552x over previous
import functools

import jax
import jax.numpy as jnp
from jax import lax
from jax.experimental import pallas as pl
from jax.experimental.pallas import tpu as pltpu
from jax.experimental.pallas import tpu_sc as plsc

H = 128
CE = 64
NP = 10240
EP = 327680
NSPLIT = 2
EH = EP // NSPLIT
TE = 2048
TN = 1024

NW = 32
GC = 64
GR = 4
SC = 128

_INTERPRET = False


def _sc_mesh():
    return plsc.VectorSubcoreMesh(core_axis_name="c", subcore_axis_name="s")


def _sc_gather_add(a, b, src, dst):
    eph = src.shape[0]
    epw = eph // NW
    gm = epw // GC

    @functools.partial(
        pl.kernel, mesh=_sc_mesh(),
        out_type=jax.ShapeDtypeStruct((eph, H), jnp.float32),
        scratch_types=[
            pltpu.VMEM((epw,), jnp.int32),
            pltpu.VMEM((epw,), jnp.int32),
        ] + [pltpu.VMEM((GC, H), jnp.float32)] * (2 * GR)
          + [pltpu.SemaphoreType.DMA] * (2 * GR),
    )
    def k(a_hbm, b_hbm, src_hbm, dst_hbm, g_hbm, si, di, *bufs):
        bas = bufs[0:GR]
        bbs = bufs[GR:2 * GR]
        gsem = bufs[2 * GR:3 * GR]
        ssem = bufs[3 * GR:4 * GR]
        wid = lax.axis_index("s") * 2 + lax.axis_index("c")
        base = wid * epw
        pltpu.sync_copy(src_hbm.at[pl.ds(base, epw)], si)
        pltpu.sync_copy(dst_hbm.at[pl.ds(base, epw)], di)

        def fire(m, q):
            pltpu.make_async_copy(
                a_hbm.at[si.at[pl.ds(m * GC, GC)]], bas[q], gsem[q]).start()
            pltpu.make_async_copy(
                b_hbm.at[di.at[pl.ds(m * GC, GC)]], bbs[q], gsem[q]).start()

        def gwait(m, q):
            pltpu.make_async_copy(
                a_hbm.at[si.at[pl.ds(m * GC, GC)]], bas[q], gsem[q]).wait()
            pltpu.make_async_copy(
                b_hbm.at[di.at[pl.ds(m * GC, GC)]], bbs[q], gsem[q]).wait()

        def store_desc(m, q):
            return pltpu.make_async_copy(
                bas[q], g_hbm.at[pl.ds(base + m * GC, GC), :], ssem[q])

        for mm in range(2):
            fire(mm, mm)

        def outer(o, carry):
            for p in range(GR):
                m = o * GR + p
                q = (p + 2) % GR

                @pl.when(m >= 2)
                def _():
                    store_desc(m - 2, q).wait()

                @pl.when(m + 2 < gm)
                def _():
                    fire(m + 2, q)

                gwait(m, p)

                @plsc.parallel_loop(0, GC, 1, unroll=4)
                def radd(r, _p=p):
                    for j in range(H // 16):
                        s = pl.ds(j * 16, 16)
                        bas[_p][r, s] = bas[_p][r, s] + bbs[_p][r, s]

                store_desc(m, p).start()
            return carry

        lax.fori_loop(0, gm // GR, outer, 0)
        for mm in (gm - 2, gm - 1):
            store_desc(mm, mm % GR).wait()

    return k(a, b, src, dst)


def _sc_scatter_add(h, dst3, zeros_hbm):
    eph = h.shape[0]
    epw = eph // NW
    sm = epw // SC

    @functools.partial(
        pl.kernel, mesh=_sc_mesh(),
        out_type=jax.ShapeDtypeStruct((2, NP, H), jnp.float32),
        scratch_types=[
            pltpu.VMEM((sm, SC), jnp.int32),
            pltpu.VMEM((SC, H), jnp.float32),
            pltpu.VMEM((SC, H), jnp.float32),
            pltpu.VMEM_SHARED((NP, H), jnp.float32),
            pltpu.SemaphoreType.DMA,
            pltpu.SemaphoreType.DMA,
            pltpu.SemaphoreType.DMA,
            pltpu.SemaphoreType.DMA,
        ],
    )
    def k(h_hbm, dst_hbm, z_hbm, out_hbm, di2, bh0, bh1, acc, hs0, hs1, ss0, ss1):
        bhs = (bh0, bh1)
        hsem = (hs0, hs1)
        ssem = (ss0, ss1)
        ci = lax.axis_index("c")
        sj = lax.axis_index("s")
        wid = sj * 2 + ci
        rows = NP // 16
        r0 = sj * rows
        pltpu.sync_copy(z_hbm.at[pl.ds(r0, rows), :], acc.at[pl.ds(r0, rows), :])
        pltpu.sync_copy(dst_hbm.at[wid], di2)
        base = wid * epw

        def hload_desc(m, q):
            return pltpu.make_async_copy(
                h_hbm.at[pl.ds(base + m * SC, SC), :], bhs[q], hsem[q])

        def sadd_desc(m, q):
            return pltpu.make_async_copy(bhs[q], acc.at[di2.at[m]], ssem[q])

        plsc.subcore_barrier()
        hload_desc(0, 0).start()

        def outer(o, carry):
            for p in range(2):
                m = o * 2 + p

                @pl.when(m >= 1)
                def _():
                    sadd_desc(m - 1, 1 - p).wait()

                @pl.when(m + 1 < sm)
                def _():
                    hload_desc(m + 1, 1 - p).start()

                hload_desc(m, p).wait()
                sadd_desc(m, p).start(add=True)
            return carry

        lax.fori_loop(0, sm // 2, outer, 0)
        sadd_desc(sm - 1, 1).wait()
        plsc.subcore_barrier()
        pltpu.sync_copy(acc.at[pl.ds(r0, rows), :],
                        out_hbm.at[ci, pl.ds(r0, rows), :])

    return k(h, dst3, zeros_hbm)


def _dot(a, b):
    return jnp.dot(a, b, preferred_element_type=jnp.float32)


def _encode_body(lg_ref, gw1_ref, gb1_ref, nf_ref):
    nfh = jnp.maximum(_dot(lg_ref[...], gw1_ref[...]) + gb1_ref[...], 0.0)
    nf_ref[...] = jnp.concatenate([nfh, jnp.zeros_like(nfh)], axis=-1)


def _ab_body(nf_ref, ws_ref, wd_ref, a_ref, b_ref):
    nf = nf_ref[...]
    a_ref[...] = _dot(nf, ws_ref[...])
    b_ref[...] = _dot(nf, wd_ref[...])


def _edge_body(g_ref, ea_ref, ew1_ref, eb1_ref, ew2_ref, eb2_ref,
               me_ref, mb1_ref, mw2_ref, mb2_ref, h_ref):
    ef = jnp.maximum(_dot(ea_ref[...], ew1_ref[...]) + eb1_ref[...], 0.0)
    ef = _dot(ef, ew2_ref[...]) + eb2_ref[...]
    pre = g_ref[...] + _dot(ef, me_ref[...]) + mb1_ref[...]
    h_ref[...] = _dot(jnp.maximum(pre, 0.0), mw2_ref[...]) + mb2_ref[...]


def _node_body(nf_ref, g0_ref, g1_ref, g2_ref, g3_ref, uwn_ref, uwa_ref,
               ub1_ref, uw2_ref, ub2_ref, lng_ref, lnb_ref, out_ref):
    nf = nf_ref[...]
    agg = (g0_ref[...] + g1_ref[...]) + (g2_ref[...] + g3_ref[...])
    pre = _dot(nf, uwn_ref[...]) + _dot(agg, uwa_ref[...]) + ub1_ref[...]
    upd = _dot(jnp.maximum(pre, 0.0), uw2_ref[...]) + ub2_ref[...]
    x = nf + upd
    mean = jnp.mean(x, axis=-1, keepdims=True)
    var = jnp.mean((x - mean) ** 2, axis=-1, keepdims=True)
    out_ref[...] = (x - mean) * jax.lax.rsqrt(var + 1e-5) * lng_ref[...] + lnb_ref[...]


def _out_body(nf_ref, ow1_ref, ob1_ref, ow2_ref, ob2_ref, out_ref):
    h = jnp.maximum(_dot(nf_ref[...], ow1_ref[...]) + ob1_ref[...], 0.0)
    out_ref[...] = _dot(h, ow2_ref[...]) + ob2_ref[...]


def _row_spec(t, k):
    return pl.BlockSpec((t, k), lambda i: (i, 0))


def _full_spec(shape):
    return pl.BlockSpec(shape, lambda i: (0,) * len(shape))


def _tc_call(body, nrows, trows, ins, out_shapes, out_dtype=jnp.float32):
    in_specs = []
    for x in ins:
        if x.ndim == 2 and x.shape[0] == nrows:
            in_specs.append(_row_spec(trows, x.shape[1]))
        else:
            in_specs.append(_full_spec(x.shape))
    if isinstance(out_shapes, tuple):
        out_shape = tuple(jax.ShapeDtypeStruct((nrows, k), out_dtype)
                          for k in out_shapes)
        out_specs = tuple(_row_spec(trows, k) for k in out_shapes)
    else:
        out_shape = jax.ShapeDtypeStruct((nrows, out_shapes), out_dtype)
        out_specs = _row_spec(trows, out_shapes)
    return pl.pallas_call(
        body,
        grid=(nrows // trows,),
        in_specs=in_specs,
        out_specs=out_specs,
        out_shape=out_shape,
        interpret=_INTERPRET,
    )(*ins)


def _row(v):
    return v.reshape(1, -1)


def kernel(edge_index, edge_attr, local_geometry, num_nodes, params):
    p = params
    N = local_geometry.shape[0]
    E = edge_attr.shape[0]

    src = edge_index[0].astype(jnp.int32)
    dst = edge_index[1].astype(jnp.int32)
    src_p = jnp.concatenate([src, jnp.zeros((EP - E,), jnp.int32)])
    dst_p = jnp.concatenate([dst, jnp.full((EP - E,), N, jnp.int32)])
    ea_p = jnp.zeros((EP, 8), jnp.float32).at[:E, :3].set(edge_attr)
    lg_p = jnp.zeros((NP, 8), jnp.float32).at[:N, :5].set(local_geometry)

    gw1 = jnp.zeros((8, H // 2), jnp.float32).at[:5].set(p['gw1'])
    ew1 = jnp.zeros((8, CE), jnp.float32).at[:3].set(p['ew1'])

    nf = _tc_call(_encode_body, NP, TN, (lg_p, gw1, _row(p['gb1'])), H)
    zeros_np = jnp.zeros((NP, H), jnp.float32)

    halves = []
    for hi in range(NSPLIT):
        sl = slice(hi * EH, (hi + 1) * EH)
        halves.append((src_p[sl], dst_p[sl],
                       dst_p[sl].reshape(NW, EH // NW // SC, SC), ea_p[sl]))

    for li, L in enumerate(p['layers']):
        ws, wd, me = L['mw1'][:H], L['mw1'][H:2 * H], L['mw1'][2 * H:]
        a, b = _tc_call(_ab_body, NP, TN, (nf, ws, wd), (H, H))

        gs = [_sc_gather_add(a, b, sh, dh) for sh, dh, _, _ in halves]
        hs = [_tc_call(_edge_body, EH, TE,
                       (g, eah, ew1, _row(p['eb1']), p['ew2'], _row(p['eb2']),
                        me, _row(L['mb1']), L['mw2'], _row(L['mb2'])), H)
              for g, (_, _, _, eah) in zip(gs, halves)]
        ps = [_sc_scatter_add(h, d3, zeros_np)
              for h, (_, _, d3, _) in zip(hs, halves)]

        aggs = tuple(pp[i] for pp in ps for i in range(2))
        nf = _tc_call(_node_body, NP, TN,
                      (nf,) + aggs +
                      (L['uw1'][:H], L['uw1'][H:], _row(L['ub1']),
                       L['uw2'], _row(L['ub2']),
                       _row(L['ln_g']), _row(L['ln_b'])), H)

    out = _tc_call(_out_body, NP, TN,
                   (nf, p['ow1'], _row(p['ob1']), p['ow2'], _row(p['ob2'])), H)
    return out[:N]


# speedup vs baseline: 1.0558x; 1.0006x over previous
import functools

import jax
import jax.numpy as jnp
from jax import lax
from jax.experimental import pallas as pl
from jax.experimental.pallas import tpu as pltpu
from jax.experimental.pallas import tpu_sc as plsc

H = 128
CE = 64
NP = 10240
EP = 327680
NSPLIT = 2
EH = EP // NSPLIT
TE = 2048
TN = 1024

NW = 32
GC = 64
GR = 4
SC = 128


def _sc_mesh():
    return plsc.VectorSubcoreMesh(core_axis_name="c", subcore_axis_name="s")


def _sc_gather_add(a, b, src, dst):
    eph = src.shape[0]
    epw = eph // NW
    gm = epw // GC

    @functools.partial(
        pl.kernel, mesh=_sc_mesh(),
        out_type=jax.ShapeDtypeStruct((eph, H), jnp.float32),
        scratch_types=[
            pltpu.VMEM((epw,), jnp.int32),
            pltpu.VMEM((epw,), jnp.int32),
        ] + [pltpu.VMEM((GC, H), jnp.float32)] * (2 * GR)
          + [pltpu.SemaphoreType.DMA] * (2 * GR),
    )
    def k(a_hbm, b_hbm, src_hbm, dst_hbm, g_hbm, si, di, *bufs):
        bas = bufs[0:GR]
        bbs = bufs[GR:2 * GR]
        gsem = bufs[2 * GR:3 * GR]
        ssem = bufs[3 * GR:4 * GR]
        wid = lax.axis_index("s") * 2 + lax.axis_index("c")
        base = wid * epw
        pltpu.sync_copy(src_hbm.at[pl.ds(base, epw)], si)
        pltpu.sync_copy(dst_hbm.at[pl.ds(base, epw)], di)

        def fire(m, q):
            pltpu.make_async_copy(
                a_hbm.at[si.at[pl.ds(m * GC, GC)]], bas[q], gsem[q]).start()
            pltpu.make_async_copy(
                b_hbm.at[di.at[pl.ds(m * GC, GC)]], bbs[q], gsem[q]).start()

        def gwait(m, q):
            pltpu.make_async_copy(
                a_hbm.at[si.at[pl.ds(m * GC, GC)]], bas[q], gsem[q]).wait()
            pltpu.make_async_copy(
                b_hbm.at[di.at[pl.ds(m * GC, GC)]], bbs[q], gsem[q]).wait()

        def store_desc(m, q):
            return pltpu.make_async_copy(
                bas[q], g_hbm.at[pl.ds(base + m * GC, GC), :], ssem[q])

        for mm in range(2):
            fire(mm, mm)

        def outer(o, carry):
            for p in range(GR):
                m = o * GR + p
                q = (p + 2) % GR

                @pl.when(m >= 2)
                def _():
                    store_desc(m - 2, q).wait()

                @pl.when(m + 2 < gm)
                def _():
                    fire(m + 2, q)

                gwait(m, p)

                @plsc.parallel_loop(0, GC, 1, unroll=4)
                def radd(r, _p=p):
                    for j in range(H // 16):
                        s = pl.ds(j * 16, 16)
                        bas[_p][r, s] = bas[_p][r, s] + bbs[_p][r, s]

                store_desc(m, p).start()
            return carry

        lax.fori_loop(0, gm // GR, outer, 0)
        for mm in (gm - 2, gm - 1):
            store_desc(mm, mm % GR).wait()

    return k(a, b, src, dst)


def _sc_scatter_add(h, dst3, zeros_hbm):
    eph = h.shape[0]
    epw = eph // NW
    sm = epw // SC

    @functools.partial(
        pl.kernel, mesh=_sc_mesh(),
        out_type=jax.ShapeDtypeStruct((2, NP, H), jnp.float32),
        scratch_types=[
            pltpu.VMEM((sm, SC), jnp.int32),
            pltpu.VMEM((SC, H), jnp.float32),
            pltpu.VMEM((SC, H), jnp.float32),
            pltpu.VMEM_SHARED((NP, H), jnp.float32),
            pltpu.SemaphoreType.DMA,
            pltpu.SemaphoreType.DMA,
            pltpu.SemaphoreType.DMA,
            pltpu.SemaphoreType.DMA,
        ],
    )
    def k(h_hbm, dst_hbm, z_hbm, out_hbm, di2, bh0, bh1, acc, hs0, hs1, ss0, ss1):
        bhs = (bh0, bh1)
        hsem = (hs0, hs1)
        ssem = (ss0, ss1)
        ci = lax.axis_index("c")
        sj = lax.axis_index("s")
        wid = sj * 2 + ci
        rows = NP // 16
        r0 = sj * rows
        pltpu.sync_copy(z_hbm.at[pl.ds(r0, rows), :], acc.at[pl.ds(r0, rows), :])
        pltpu.sync_copy(dst_hbm.at[wid], di2)
        base = wid * epw

        def hload_desc(m, q):
            return pltpu.make_async_copy(
                h_hbm.at[pl.ds(base + m * SC, SC), :], bhs[q], hsem[q])

        def sadd_desc(m, q):
            return pltpu.make_async_copy(bhs[q], acc.at[di2.at[m]], ssem[q])

        plsc.subcore_barrier()
        hload_desc(0, 0).start()

        def outer(o, carry):
            for p in range(2):
                m = o * 2 + p

                @pl.when(m >= 1)
                def _():
                    sadd_desc(m - 1, 1 - p).wait()

                @pl.when(m + 1 < sm)
                def _():
                    hload_desc(m + 1, 1 - p).start()

                hload_desc(m, p).wait()
                sadd_desc(m, p).start(add=True)
            return carry

        lax.fori_loop(0, sm // 2, outer, 0)
        sadd_desc(sm - 1, 1).wait()
        plsc.subcore_barrier()
        pltpu.sync_copy(acc.at[pl.ds(r0, rows), :],
                        out_hbm.at[ci, pl.ds(r0, rows), :])

    return k(h, dst3, zeros_hbm)


def _dot(a, b):
    return jnp.dot(a, b, preferred_element_type=jnp.float32)


def _encode_body(lg_ref, gw1_ref, gb1_ref, nf_ref):
    nfh = jnp.maximum(_dot(lg_ref[...], gw1_ref[...]) + gb1_ref[...], 0.0)
    nf_ref[...] = jnp.concatenate([nfh, jnp.zeros_like(nfh)], axis=-1)


def _ab_body(nf_ref, ws_ref, wd_ref, a_ref, b_ref):
    nf = nf_ref[...]
    a_ref[...] = _dot(nf, ws_ref[...])
    b_ref[...] = _dot(nf, wd_ref[...])


def _edge_body(g_ref, ea_ref, ew1_ref, eb1_ref, ew2_ref, eb2_ref,
               me_ref, mb1_ref, mw2_ref, mb2_ref, h_ref):
    ef = jnp.maximum(_dot(ea_ref[...], ew1_ref[...]) + eb1_ref[...], 0.0)
    ef = _dot(ef, ew2_ref[...]) + eb2_ref[...]
    pre = g_ref[...] + _dot(ef, me_ref[...]) + mb1_ref[...]
    h_ref[...] = _dot(jnp.maximum(pre, 0.0), mw2_ref[...]) + mb2_ref[...]


def _node_body(nf_ref, g0_ref, g1_ref, g2_ref, g3_ref, uwn_ref, uwa_ref,
               ub1_ref, uw2_ref, ub2_ref, lng_ref, lnb_ref, out_ref):
    nf = nf_ref[...]
    agg = (g0_ref[...] + g1_ref[...]) + (g2_ref[...] + g3_ref[...])
    pre = _dot(nf, uwn_ref[...]) + _dot(agg, uwa_ref[...]) + ub1_ref[...]
    upd = _dot(jnp.maximum(pre, 0.0), uw2_ref[...]) + ub2_ref[...]
    x = nf + upd
    mean = jnp.mean(x, axis=-1, keepdims=True)
    var = jnp.mean((x - mean) ** 2, axis=-1, keepdims=True)
    out_ref[...] = (x - mean) * jax.lax.rsqrt(var + 1e-5) * lng_ref[...] + lnb_ref[...]


def _out_body(nf_ref, ow1_ref, ob1_ref, ow2_ref, ob2_ref, out_ref):
    h = jnp.maximum(_dot(nf_ref[...], ow1_ref[...]) + ob1_ref[...], 0.0)
    out_ref[...] = _dot(h, ow2_ref[...]) + ob2_ref[...]


def _row_spec(t, k):
    return pl.BlockSpec((t, k), lambda i: (i, 0))


def _full_spec(shape):
    return pl.BlockSpec(shape, lambda i: (0,) * len(shape))


def _tc_call(body, nrows, trows, ins, out_shapes, out_dtype=jnp.float32):
    in_specs = []
    for x in ins:
        if x.ndim == 2 and x.shape[0] == nrows:
            in_specs.append(_row_spec(trows, x.shape[1]))
        else:
            in_specs.append(_full_spec(x.shape))
    if isinstance(out_shapes, tuple):
        out_shape = tuple(jax.ShapeDtypeStruct((nrows, k), out_dtype)
                          for k in out_shapes)
        out_specs = tuple(_row_spec(trows, k) for k in out_shapes)
    else:
        out_shape = jax.ShapeDtypeStruct((nrows, out_shapes), out_dtype)
        out_specs = _row_spec(trows, out_shapes)
    return pl.pallas_call(
        body,
        grid=(nrows // trows,),
        in_specs=in_specs,
        out_specs=out_specs,
        out_shape=out_shape,
    )(*ins)


def _row(v):
    return v.reshape(1, -1)


def kernel(edge_index, edge_attr, local_geometry, num_nodes, params):
    p = params
    N = local_geometry.shape[0]
    E = edge_attr.shape[0]

    src = edge_index[0].astype(jnp.int32)
    dst = edge_index[1].astype(jnp.int32)
    src_p = jnp.concatenate([src, jnp.zeros((EP - E,), jnp.int32)])
    dst_p = jnp.concatenate([dst, jnp.full((EP - E,), N, jnp.int32)])
    ea_p = jnp.zeros((EP, 8), jnp.float32).at[:E, :3].set(edge_attr)
    lg_p = jnp.zeros((NP, 8), jnp.float32).at[:N, :5].set(local_geometry)

    gw1 = jnp.zeros((8, H // 2), jnp.float32).at[:5].set(p['gw1'])
    ew1 = jnp.zeros((8, CE), jnp.float32).at[:3].set(p['ew1'])

    nf = _tc_call(_encode_body, NP, TN, (lg_p, gw1, _row(p['gb1'])), H)
    zeros_np = jnp.zeros((NP, H), jnp.float32)

    halves = []
    for hi in range(NSPLIT):
        sl = slice(hi * EH, (hi + 1) * EH)
        halves.append((src_p[sl], dst_p[sl],
                       dst_p[sl].reshape(NW, EH // NW // SC, SC), ea_p[sl]))

    for li, L in enumerate(p['layers']):
        ws, wd, me = L['mw1'][:H], L['mw1'][H:2 * H], L['mw1'][2 * H:]
        a, b = _tc_call(_ab_body, NP, TN, (nf, ws, wd), (H, H))

        gs = [_sc_gather_add(a, b, sh, dh) for sh, dh, _, _ in halves]
        hs = [_tc_call(_edge_body, EH, TE,
                       (g, eah, ew1, _row(p['eb1']), p['ew2'], _row(p['eb2']),
                        me, _row(L['mb1']), L['mw2'], _row(L['mb2'])), H)
              for g, (_, _, _, eah) in zip(gs, halves)]
        ps = [_sc_scatter_add(h, d3, zeros_np)
              for h, (_, _, d3, _) in zip(hs, halves)]

        aggs = tuple(pp[i] for pp in ps for i in range(2))
        nf = _tc_call(_node_body, NP, TN,
                      (nf,) + aggs +
                      (L['uw1'][:H], L['uw1'][H:], _row(L['ub1']),
                       L['uw2'], _row(L['ub2']),
                       _row(L['ln_g']), _row(L['ln_b'])), H)

    out = _tc_call(_out_body, NP, TN,
                   (nf, p['ow1'], _row(p['ob1']), p['ow2'], _row(p['ob2'])), H)
    return out[:N]
